# baseline (speedup 1.0000x reference)
import functools

import jax
import jax.numpy as jnp
from jax import lax
from jax.experimental import pallas as pl
from jax.experimental.pallas import tpu as pltpu
from jax.experimental.pallas import tpu_sc as plsc

VOCAB = 50257
SEQ = 2048
EMBED = 768
BATCH = 4

_INFO = plsc.get_sparse_core_info()
NC = _INFO.num_cores
NS = _INFO.num_subcores
L = _INFO.num_lanes
NW = NC * NS
S_PER_W = SEQ // NW
VPR = EMBED // L


@functools.partial(
    pl.kernel,
    out_type=jax.ShapeDtypeStruct((BATCH * SEQ, EMBED), jnp.float32),
    mesh=plsc.VectorSubcoreMesh(core_axis_name="c", subcore_axis_name="s"),
    scratch_types=[
        pltpu.VMEM((S_PER_W,), jnp.int32),
        pltpu.VMEM((S_PER_W, EMBED), jnp.float32),
        pltpu.VMEM((S_PER_W, EMBED), jnp.float32),
        pltpu.SemaphoreType.DMA,
    ],
)
def _emb_kernel(ids_hbm, table_hbm, pos_hbm, out_hbm, idx_v, pos_v, rows_v, sem):
    wid = lax.axis_index("s") * NC + lax.axis_index("c")
    s_base = wid * S_PER_W

    pltpu.sync_copy(pos_hbm.at[pl.ds(s_base, S_PER_W)], pos_v)

    for b in range(BATCH):
        off = b * SEQ + s_base
        pltpu.sync_copy(ids_hbm.at[pl.ds(off, S_PER_W)], idx_v)
        pltpu.async_copy(table_hbm.at[idx_v], rows_v, sem).wait()

        def add_row(i, _):
            for j in range(VPR):
                plsc.addupdate(
                    rows_v.at[i, pl.ds(j * L, L)],
                    pos_v[i, pl.ds(j * L, L)],
                )
            return 0

        lax.fori_loop(0, S_PER_W, add_row, 0)

        pltpu.sync_copy(rows_v, out_hbm.at[pl.ds(off, S_PER_W)])


def kernel(input_ids, token_embeddings, position_embeddings):
    ids = input_ids.reshape(-1).astype(jnp.int32)
    out = _emb_kernel(ids, token_embeddings, position_embeddings)
    return out.reshape(BATCH, SEQ, EMBED)
